# baseline (speedup 1.0000x reference)
import functools

import jax
import jax.numpy as jnp
from jax import lax
from jax.experimental import pallas as pl
from jax.experimental.pallas import tpu as pltpu
from jax.experimental.pallas import tpu_sc as plsc

_B = 32
_D_IN = 2048
_D_SAE = 32768
_K = 64
_BLK_N = 1024

_NCHUNK = _D_SAE // 16
_SHIFTS = (23, 14, 5, 0)
_WIDTHS = (9, 9, 9, 5)


def _encode_body(x_ref, bdec_ref, w_ref, benc_ref, out_ref):
    xm = x_ref[...] - bdec_ref[...]
    y = jnp.dot(xm, w_ref[...], preferred_element_type=jnp.float32)
    out_ref[...] = jnp.maximum(y + benc_ref[...], 0.0)


def _encode(x, W_enc, b_enc, b_dec):
    grid = (_D_SAE // _BLK_N,)
    return pl.pallas_call(
        _encode_body,
        grid=grid,
        in_specs=[
            pl.BlockSpec((_B, _D_IN), lambda i: (0, 0)),
            pl.BlockSpec((1, _D_IN), lambda i: (0, 0)),
            pl.BlockSpec((_D_IN, _BLK_N), lambda i: (0, i)),
            pl.BlockSpec((1, _BLK_N), lambda i: (0, i)),
        ],
        out_specs=pl.BlockSpec((_B, _BLK_N), lambda i: (0, i)),
        out_shape=jax.ShapeDtypeStruct((_B, _D_SAE), jnp.float32),
        compiler_params=pltpu.CompilerParams(
            dimension_semantics=("arbitrary",),
        ),
    )(x, b_dec.reshape(1, _D_IN), W_enc, b_enc.reshape(1, _D_SAE))


def _scan_threshold(hist1, totals, need):
    cum = jnp.int32(0)
    c_t = jnp.int32(0)
    cum_t = jnp.int32(0)
    found = jnp.bool_(False)
    for c in reversed(range(32)):
        tc = totals[c]
        hit = jnp.logical_and(jnp.logical_not(found), cum + tc >= need)
        c_t = jnp.where(hit, jnp.int32(c), c_t)
        cum_t = jnp.where(hit, cum, cum_t)
        found = jnp.logical_or(found, hit)
        cum = cum + tc
    cum2 = cum_t
    bkt = jnp.int32(0)
    gt_b = jnp.int32(0)
    cnt_b = jnp.int32(0)
    found2 = jnp.bool_(False)
    for j in reversed(range(16)):
        cj = hist1[c_t * 16 + j]
        hit = jnp.logical_and(jnp.logical_not(found2), cum2 + cj >= need)
        bkt = jnp.where(hit, c_t * 16 + j, bkt)
        gt_b = jnp.where(hit, cum2, gt_b)
        cnt_b = jnp.where(hit, cj, cnt_b)
        found2 = jnp.logical_or(found2, hit)
        cum2 = cum2 + cj
    return bkt, gt_b, cnt_b


def _popcnt(m):
    return plsc.all_reduce_population_count(m)[0]


def _topk_decode_body(acts_hbm, wdec_hbm, bdec_hbm, out_hbm,
                      row_v, hist2, acc_idx, acc_val,
                      rows2, acc_v, shist, totals, st,
                      sem0, sem1, sem2, sem3):
    wid = lax.axis_index("s") * 2 + lax.axis_index("c")
    pltpu.sync_copy(acts_hbm.at[wid], row_v)
    pltpu.sync_copy(bdec_hbm, acc_v)

    lanes = lax.iota(jnp.int32, 16)
    ones = jnp.ones((16,), jnp.int32)
    zeros16i = jnp.zeros((16,), jnp.int32)

    st[0] = jnp.int32(0)
    st[1] = jnp.int32(0)
    st[2] = jnp.int32(_K)
    st[3] = jnp.int32(0)

    for r in range(4):
        sh = _SHIFTS[r]
        fmask = (1 << _WIDTHS[r]) - 1

        def round_body(r=r, sh=sh, fmask=fmask):
            p_prev = st[0]
            ps_prev = st[1]
            need = st[2]
            @plsc.parallel_loop(0, 8192, step=16, unroll=8)
            def _zero(b):
                hist2[pl.ds(b, 16)] = zeros16i

            def _mk_fill(lo, hi):
                @plsc.parallel_loop(lo, hi, step=16, unroll=16)
                def _fill(i):
                    bits = lax.bitcast_convert_type(row_v[pl.ds(i, 16)],
                                                    jnp.int32)
                    f = jnp.bitwise_and(
                        jax.lax.shift_right_logical(bits, sh), fmask)
                    fl = f * 16 + lanes
                    if r == 0:
                        plsc.addupdate_scatter(hist2, [fl], ones)
                    else:
                        cand = (jax.lax.shift_right_logical(bits, ps_prev)
                                == p_prev)
                        plsc.addupdate_scatter(hist2, [fl], ones, mask=cand)

            _mk_fill(0, _D_SAE)

            @plsc.parallel_loop(0, 512, step=1, unroll=8)
            def _reduce(b):
                shist[b] = jnp.sum(hist2[pl.ds(b * 16, 16)])

            @plsc.parallel_loop(0, 32, step=1, unroll=4)
            def _tot(c):
                t = shist[c * 16]
                for l in range(1, 16):
                    t = t + shist[c * 16 + l]
                totals[c] = t

            bkt, gt_b, cnt_b = _scan_threshold(shist, totals, need)
            new_need = need - gt_b
            st[0] = jnp.bitwise_or(
                jax.lax.shift_left(p_prev, _WIDTHS[r]), bkt)
            st[1] = jnp.int32(sh)
            st[2] = new_need
            st[3] = (cnt_b == new_need).astype(jnp.int32)

        if r == 0:
            round_body()
        else:
            pl.when(st[3] == 0)(round_body)

    p_f = st[0]
    ps_f = st[1]

    q0 = jnp.zeros((16,), jnp.int32) + st[2]

    @plsc.parallel_loop(0, _D_SAE, step=16, unroll=16,
                        carry=(jnp.zeros((16,), jnp.int32), q0))
    def _collect(i, carry):
        ptr, q = carry
        v = row_v[pl.ds(i, 16)]
        bits = lax.bitcast_convert_type(v, jnp.int32)
        hb = jax.lax.shift_right_logical(bits, ps_f)
        m_gt = hb > p_f
        m_eq = hb == p_f
        ce = plsc.cumsum(m_eq.astype(jnp.int32))
        acc_eq = jnp.logical_and(m_eq, ce <= q)
        m = jnp.logical_or(m_gt, acc_eq)
        rank = ptr + plsc.cumsum(m.astype(jnp.int32)) - 1
        plsc.store_scatter(acc_val, [rank], v, mask=m)
        plsc.store_scatter(acc_idx, [rank], lanes + i, mask=m)
        return (ptr + plsc.all_reduce_population_count(m),
                q - plsc.all_reduce_population_count(acc_eq))

    sems = [sem0, sem1]
    cps = [None, None]
    cps[0] = pltpu.async_copy(
        wdec_hbm.at[acc_idx.at[pl.ds(0, 8)]], rows2.at[0], sems[0])
    for g in range(_K // 8):
        cur = g % 2
        if g + 1 < _K // 8:
            nxt = (g + 1) % 2
            cps[nxt] = pltpu.async_copy(
                wdec_hbm.at[acc_idx.at[pl.ds((g + 1) * 8, 8)]],
                rows2.at[nxt], sems[nxt])
        cps[cur].wait()
        vv = acc_val[pl.ds((g // 2) * 16, 16)]
        a = [vv[(g % 2) * 8 + j] for j in range(8)]

        @plsc.parallel_loop(0, _D_IN, step=16, unroll=8)
        def _accum(t, a=a, cur=cur):
            sl = pl.ds(t, 16)
            v = acc_v[sl]
            for j in range(8):
                v = v + a[j] * rows2[cur, j, sl]
            acc_v[sl] = v
    pltpu.sync_copy(acc_v, out_hbm.at[wid])


def _topk_decode_sc(acts, W_dec, b_dec):
    mesh = plsc.VectorSubcoreMesh(core_axis_name="c", subcore_axis_name="s")
    fn = functools.partial(
        pl.kernel,
        out_type=jax.ShapeDtypeStruct((_B, _D_IN), jnp.float32),
        mesh=mesh,
        scratch_types=[
            pltpu.VMEM((_D_SAE,), jnp.float32),
            pltpu.VMEM((8192,), jnp.int32),
            pltpu.VMEM((_K + 16,), jnp.int32),
            pltpu.VMEM((_K + 16,), jnp.float32),
            pltpu.VMEM((2, 8, _D_IN), jnp.float32),
            pltpu.VMEM((_D_IN,), jnp.float32),
            pltpu.SMEM((512,), jnp.int32),
            pltpu.SMEM((32,), jnp.int32),
            pltpu.SMEM((8,), jnp.int32),
            pltpu.SemaphoreType.DMA,
            pltpu.SemaphoreType.DMA,
            pltpu.SemaphoreType.DMA,
            pltpu.SemaphoreType.DMA,
        ],
        compiler_params=pltpu.CompilerParams(needs_layout_passes=False),
    )(_topk_decode_body)
    return fn(acts, W_dec, b_dec)


def kernel(x, W_enc, W_dec, b_enc, b_dec):
    acts = _encode(x, W_enc, b_enc, b_dec)
    return _topk_decode_sc(acts, W_dec, b_dec)


# speedup vs baseline: 1.0124x; 1.0124x over previous
import functools

import jax
import jax.numpy as jnp
from jax import lax
from jax.experimental import pallas as pl
from jax.experimental.pallas import tpu as pltpu
from jax.experimental.pallas import tpu_sc as plsc

_B = 32
_D_IN = 2048
_D_SAE = 32768
_K = 64
_BLK_N = 1024

_NCHUNK = _D_SAE // 16
_SHIFTS = (23, 14, 5, 0)
_WIDTHS = (9, 9, 9, 5)


def _encode_body(x_ref, bdec_ref, w_ref, benc_ref, out_ref):
    xm = x_ref[...] - bdec_ref[...]
    y = jnp.dot(xm, w_ref[...], preferred_element_type=jnp.float32)
    out_ref[...] = jnp.maximum(y + benc_ref[...], 0.0)


def _encode(x, W_enc, b_enc, b_dec):
    grid = (_D_SAE // _BLK_N,)
    return pl.pallas_call(
        _encode_body,
        grid=grid,
        in_specs=[
            pl.BlockSpec((_B, _D_IN), lambda i: (0, 0)),
            pl.BlockSpec((1, _D_IN), lambda i: (0, 0)),
            pl.BlockSpec((_D_IN, _BLK_N), lambda i: (0, i)),
            pl.BlockSpec((1, _BLK_N), lambda i: (0, i)),
        ],
        out_specs=pl.BlockSpec((_B, _BLK_N), lambda i: (0, i)),
        out_shape=jax.ShapeDtypeStruct((_B, _D_SAE), jnp.float32),
        compiler_params=pltpu.CompilerParams(
            dimension_semantics=("arbitrary",),
        ),
    )(x, b_dec.reshape(1, _D_IN), W_enc, b_enc.reshape(1, _D_SAE))


def _scan_threshold(hist1, totals, need):
    cum = jnp.int32(0)
    c_t = jnp.int32(0)
    cum_t = jnp.int32(0)
    found = jnp.bool_(False)
    for c in reversed(range(32)):
        tc = totals[c]
        hit = jnp.logical_and(jnp.logical_not(found), cum + tc >= need)
        c_t = jnp.where(hit, jnp.int32(c), c_t)
        cum_t = jnp.where(hit, cum, cum_t)
        found = jnp.logical_or(found, hit)
        cum = cum + tc
    cum2 = cum_t
    bkt = jnp.int32(0)
    gt_b = jnp.int32(0)
    cnt_b = jnp.int32(0)
    found2 = jnp.bool_(False)
    for j in reversed(range(16)):
        cj = hist1[c_t * 16 + j]
        hit = jnp.logical_and(jnp.logical_not(found2), cum2 + cj >= need)
        bkt = jnp.where(hit, c_t * 16 + j, bkt)
        gt_b = jnp.where(hit, cum2, gt_b)
        cnt_b = jnp.where(hit, cj, cnt_b)
        found2 = jnp.logical_or(found2, hit)
        cum2 = cum2 + cj
    return bkt, gt_b, cnt_b


def _popcnt(m):
    return plsc.all_reduce_population_count(m)[0]


def _topk_decode_body(acts_hbm, wdec_hbm, bdec_hbm, out_hbm,
                      row_v, hist2, acc_idx, acc_val,
                      rows2, acc_v, shist, totals, st,
                      sem0, sem1, sem2, sem3):
    wid = lax.axis_index("s") * 2 + lax.axis_index("c")
    pltpu.sync_copy(acts_hbm.at[wid], row_v)
    pltpu.sync_copy(bdec_hbm, acc_v)

    lanes = lax.iota(jnp.int32, 16)
    ones = jnp.ones((16,), jnp.int32)
    zeros16i = jnp.zeros((16,), jnp.int32)

    st[0] = jnp.int32(0)
    st[1] = jnp.int32(0)
    st[2] = jnp.int32(_K)
    st[3] = jnp.int32(0)

    for r in range(4):
        sh = _SHIFTS[r]
        fmask = (1 << _WIDTHS[r]) - 1

        def round_body(r=r, sh=sh, fmask=fmask):
            p_prev = st[0]
            ps_prev = st[1]
            need = st[2]
            @plsc.parallel_loop(0, 8192, step=16, unroll=8)
            def _zero(b):
                hist2[pl.ds(b, 16)] = zeros16i

            def _mk_fill(lo, hi):
                @plsc.parallel_loop(lo, hi, step=16, unroll=4)
                def _fill(i):
                    bits = lax.bitcast_convert_type(row_v[pl.ds(i, 16)],
                                                    jnp.int32)
                    f = jnp.bitwise_and(
                        jax.lax.shift_right_logical(bits, sh), fmask)
                    fl = f * 16 + lanes
                    if r == 0:
                        plsc.addupdate_scatter(hist2, [fl], ones)
                    else:
                        cand = (jax.lax.shift_right_logical(bits, ps_prev)
                                == p_prev)
                        plsc.addupdate_scatter(hist2, [fl], ones, mask=cand)

            _mk_fill(0, _D_SAE)

            @plsc.parallel_loop(0, 512, step=1, unroll=8)
            def _reduce(b):
                shist[b] = jnp.sum(hist2[pl.ds(b * 16, 16)])

            @plsc.parallel_loop(0, 32, step=1, unroll=4)
            def _tot(c):
                t = shist[c * 16]
                for l in range(1, 16):
                    t = t + shist[c * 16 + l]
                totals[c] = t

            bkt, gt_b, cnt_b = _scan_threshold(shist, totals, need)
            new_need = need - gt_b
            st[0] = jnp.bitwise_or(
                jax.lax.shift_left(p_prev, _WIDTHS[r]), bkt)
            st[1] = jnp.int32(sh)
            st[2] = new_need
            st[3] = (cnt_b == new_need).astype(jnp.int32)

        if r == 0:
            round_body()
        else:
            pl.when(st[3] == 0)(round_body)

    p_f = st[0]
    ps_f = st[1]

    q0 = jnp.zeros((16,), jnp.int32) + st[2]

    @plsc.parallel_loop(0, _D_SAE, step=16, unroll=4,
                        carry=(jnp.zeros((16,), jnp.int32), q0))
    def _collect(i, carry):
        ptr, q = carry
        v = row_v[pl.ds(i, 16)]
        bits = lax.bitcast_convert_type(v, jnp.int32)
        hb = jax.lax.shift_right_logical(bits, ps_f)
        m_gt = hb > p_f
        m_eq = hb == p_f
        ce = plsc.cumsum(m_eq.astype(jnp.int32))
        acc_eq = jnp.logical_and(m_eq, ce <= q)
        m = jnp.logical_or(m_gt, acc_eq)
        rank = ptr + plsc.cumsum(m.astype(jnp.int32)) - 1
        plsc.store_scatter(acc_val, [rank], v, mask=m)
        plsc.store_scatter(acc_idx, [rank], lanes + i, mask=m)
        return (ptr + plsc.all_reduce_population_count(m),
                q - plsc.all_reduce_population_count(acc_eq))

    sems = [sem0, sem1]
    cps = [None, None]
    cps[0] = pltpu.async_copy(
        wdec_hbm.at[acc_idx.at[pl.ds(0, 8)]], rows2.at[0], sems[0])
    for g in range(_K // 8):
        cur = g % 2
        if g + 1 < _K // 8:
            nxt = (g + 1) % 2
            cps[nxt] = pltpu.async_copy(
                wdec_hbm.at[acc_idx.at[pl.ds((g + 1) * 8, 8)]],
                rows2.at[nxt], sems[nxt])
        cps[cur].wait()
        vv = acc_val[pl.ds((g // 2) * 16, 16)]
        a = [vv[(g % 2) * 8 + j] for j in range(8)]

        @plsc.parallel_loop(0, _D_IN, step=16, unroll=8)
        def _accum(t, a=a, cur=cur):
            sl = pl.ds(t, 16)
            v = acc_v[sl]
            for j in range(8):
                v = v + a[j] * rows2[cur, j, sl]
            acc_v[sl] = v
    pltpu.sync_copy(acc_v, out_hbm.at[wid])


def _topk_decode_sc(acts, W_dec, b_dec):
    mesh = plsc.VectorSubcoreMesh(core_axis_name="c", subcore_axis_name="s")
    fn = functools.partial(
        pl.kernel,
        out_type=jax.ShapeDtypeStruct((_B, _D_IN), jnp.float32),
        mesh=mesh,
        scratch_types=[
            pltpu.VMEM((_D_SAE,), jnp.float32),
            pltpu.VMEM((8192,), jnp.int32),
            pltpu.VMEM((_K + 16,), jnp.int32),
            pltpu.VMEM((_K + 16,), jnp.float32),
            pltpu.VMEM((2, 8, _D_IN), jnp.float32),
            pltpu.VMEM((_D_IN,), jnp.float32),
            pltpu.SMEM((512,), jnp.int32),
            pltpu.SMEM((32,), jnp.int32),
            pltpu.SMEM((8,), jnp.int32),
            pltpu.SemaphoreType.DMA,
            pltpu.SemaphoreType.DMA,
            pltpu.SemaphoreType.DMA,
            pltpu.SemaphoreType.DMA,
        ],
        compiler_params=pltpu.CompilerParams(needs_layout_passes=False),
    )(_topk_decode_body)
    return fn(acts, W_dec, b_dec)


def kernel(x, W_enc, W_dec, b_enc, b_dec):
    acts = _encode(x, W_enc, b_enc, b_dec)
    return _topk_decode_sc(acts, W_dec, b_dec)


# speedup vs baseline: 1.0340x; 1.0214x over previous
import functools

import jax
import jax.numpy as jnp
from jax import lax
from jax.experimental import pallas as pl
from jax.experimental.pallas import tpu as pltpu
from jax.experimental.pallas import tpu_sc as plsc

_B = 32
_D_IN = 2048
_D_SAE = 32768
_K = 64
_BLK_N = 1024

_NCHUNK = _D_SAE // 16
_SHIFTS = (23, 14, 5, 0)
_WIDTHS = (9, 9, 9, 5)


def _encode_body(x_ref, bdec_ref, w_ref, benc_ref, out_ref):
    xm = x_ref[...] - bdec_ref[...]
    y = jnp.dot(xm, w_ref[...], preferred_element_type=jnp.float32)
    out_ref[...] = jnp.maximum(y + benc_ref[...], 0.0)


def _encode(x, W_enc, b_enc, b_dec):
    grid = (_D_SAE // _BLK_N,)
    return pl.pallas_call(
        _encode_body,
        grid=grid,
        in_specs=[
            pl.BlockSpec((_B, _D_IN), lambda i: (0, 0)),
            pl.BlockSpec((1, _D_IN), lambda i: (0, 0)),
            pl.BlockSpec((_D_IN, _BLK_N), lambda i: (0, i)),
            pl.BlockSpec((1, _BLK_N), lambda i: (0, i)),
        ],
        out_specs=pl.BlockSpec((_B, _BLK_N), lambda i: (0, i)),
        out_shape=jax.ShapeDtypeStruct((_B, _D_SAE), jnp.float32),
        compiler_params=pltpu.CompilerParams(
            dimension_semantics=("arbitrary",),
        ),
    )(x, b_dec.reshape(1, _D_IN), W_enc, b_enc.reshape(1, _D_SAE))


def _scan_threshold(hist1, totals, need):
    cum = jnp.int32(0)
    c_t = jnp.int32(0)
    cum_t = jnp.int32(0)
    found = jnp.bool_(False)
    for c in reversed(range(32)):
        tc = totals[c]
        hit = jnp.logical_and(jnp.logical_not(found), cum + tc >= need)
        c_t = jnp.where(hit, jnp.int32(c), c_t)
        cum_t = jnp.where(hit, cum, cum_t)
        found = jnp.logical_or(found, hit)
        cum = cum + tc
    cum2 = cum_t
    bkt = jnp.int32(0)
    gt_b = jnp.int32(0)
    cnt_b = jnp.int32(0)
    found2 = jnp.bool_(False)
    for j in reversed(range(16)):
        cj = hist1[c_t * 16 + j]
        hit = jnp.logical_and(jnp.logical_not(found2), cum2 + cj >= need)
        bkt = jnp.where(hit, c_t * 16 + j, bkt)
        gt_b = jnp.where(hit, cum2, gt_b)
        cnt_b = jnp.where(hit, cj, cnt_b)
        found2 = jnp.logical_or(found2, hit)
        cum2 = cum2 + cj
    return bkt, gt_b, cnt_b


def _popcnt(m):
    return plsc.all_reduce_population_count(m)[0]


def _topk_decode_body(acts_hbm, wdec_hbm, bdec_hbm, out_hbm,
                      row_v, hist2, acc_idx, acc_val,
                      rows2, acc_v, shist, totals, st,
                      sem0, sem1, sem2, sem3):
    wid = lax.axis_index("s") * 2 + lax.axis_index("c")
    pltpu.sync_copy(acts_hbm.at[wid], row_v)
    pltpu.sync_copy(bdec_hbm, acc_v)

    lanes = lax.iota(jnp.int32, 16)
    ones = jnp.ones((16,), jnp.int32)
    zeros16i = jnp.zeros((16,), jnp.int32)

    st[0] = jnp.int32(0)
    st[1] = jnp.int32(0)
    st[2] = jnp.int32(_K)
    st[3] = jnp.int32(0)

    for r in range(4):
        sh = _SHIFTS[r]
        fmask = (1 << _WIDTHS[r]) - 1

        def round_body(r=r, sh=sh, fmask=fmask):
            p_prev = st[0]
            ps_prev = st[1]
            need = st[2]
            @plsc.parallel_loop(0, 8192, step=16, unroll=8)
            def _zero(b):
                hist2[pl.ds(b, 16)] = zeros16i

            def _mk_fill(lo, hi):
                @plsc.parallel_loop(lo, hi, step=16, unroll=8)
                def _fill(i):
                    bits = lax.bitcast_convert_type(row_v[pl.ds(i, 16)],
                                                    jnp.int32)
                    f = jnp.bitwise_and(
                        jax.lax.shift_right_logical(bits, sh), fmask)
                    fl = f * 16 + lanes
                    if r == 0:
                        plsc.addupdate_scatter(hist2, [fl], ones)
                    else:
                        cand = (jax.lax.shift_right_logical(bits, ps_prev)
                                == p_prev)
                        plsc.addupdate_scatter(hist2, [fl], ones, mask=cand)

            _mk_fill(0, _D_SAE)

            @plsc.parallel_loop(0, 512, step=1, unroll=8)
            def _reduce(b):
                shist[b] = jnp.sum(hist2[pl.ds(b * 16, 16)])

            @plsc.parallel_loop(0, 32, step=1, unroll=4)
            def _tot(c):
                t = shist[c * 16]
                for l in range(1, 16):
                    t = t + shist[c * 16 + l]
                totals[c] = t

            bkt, gt_b, cnt_b = _scan_threshold(shist, totals, need)
            new_need = need - gt_b
            st[0] = jnp.bitwise_or(
                jax.lax.shift_left(p_prev, _WIDTHS[r]), bkt)
            st[1] = jnp.int32(sh)
            st[2] = new_need
            st[3] = (cnt_b == new_need).astype(jnp.int32)

        if r == 0:
            round_body()
        else:
            pl.when(st[3] == 0)(round_body)

    p_f = st[0]
    ps_f = st[1]

    q0 = jnp.zeros((16,), jnp.int32) + st[2]

    @plsc.parallel_loop(0, _D_SAE, step=16, unroll=8,
                        carry=(jnp.zeros((16,), jnp.int32), q0))
    def _collect(i, carry):
        ptr, q = carry
        v = row_v[pl.ds(i, 16)]
        bits = lax.bitcast_convert_type(v, jnp.int32)
        hb = jax.lax.shift_right_logical(bits, ps_f)
        m_gt = hb > p_f
        m_eq = hb == p_f
        ce = plsc.cumsum(m_eq.astype(jnp.int32))
        acc_eq = jnp.logical_and(m_eq, ce <= q)
        m = jnp.logical_or(m_gt, acc_eq)
        rank = ptr + plsc.cumsum(m.astype(jnp.int32)) - 1
        plsc.store_scatter(acc_val, [rank], v, mask=m)
        plsc.store_scatter(acc_idx, [rank], lanes + i, mask=m)
        return (ptr + plsc.all_reduce_population_count(m),
                q - plsc.all_reduce_population_count(acc_eq))

    sems = [sem0, sem1]
    cps = [None, None]
    cps[0] = pltpu.async_copy(
        wdec_hbm.at[acc_idx.at[pl.ds(0, 16)]], rows2.at[0], sems[0])
    for g in range(_K // 16):
        cur = g % 2
        if g + 1 < _K // 16:
            nxt = (g + 1) % 2
            cps[nxt] = pltpu.async_copy(
                wdec_hbm.at[acc_idx.at[pl.ds((g + 1) * 16, 16)]],
                rows2.at[nxt], sems[nxt])
        cps[cur].wait()
        vv = acc_val[pl.ds(g * 16, 16)]
        a = [vv[j] for j in range(16)]

        @plsc.parallel_loop(0, _D_IN, step=16, unroll=8)
        def _accum(t, a=a, cur=cur):
            sl = pl.ds(t, 16)
            v = acc_v[sl]
            for j in range(16):
                v = v + a[j] * rows2[cur, j, sl]
            acc_v[sl] = v
    pltpu.sync_copy(acc_v, out_hbm.at[wid])


def _topk_decode_sc(acts, W_dec, b_dec):
    mesh = plsc.VectorSubcoreMesh(core_axis_name="c", subcore_axis_name="s")
    fn = functools.partial(
        pl.kernel,
        out_type=jax.ShapeDtypeStruct((_B, _D_IN), jnp.float32),
        mesh=mesh,
        scratch_types=[
            pltpu.VMEM((_D_SAE,), jnp.float32),
            pltpu.VMEM((8192,), jnp.int32),
            pltpu.VMEM((_K + 16,), jnp.int32),
            pltpu.VMEM((_K + 16,), jnp.float32),
            pltpu.VMEM((2, 16, _D_IN), jnp.float32),
            pltpu.VMEM((_D_IN,), jnp.float32),
            pltpu.SMEM((512,), jnp.int32),
            pltpu.SMEM((32,), jnp.int32),
            pltpu.SMEM((8,), jnp.int32),
            pltpu.SemaphoreType.DMA,
            pltpu.SemaphoreType.DMA,
            pltpu.SemaphoreType.DMA,
            pltpu.SemaphoreType.DMA,
        ],
        compiler_params=pltpu.CompilerParams(needs_layout_passes=False),
    )(_topk_decode_body)
    return fn(acts, W_dec, b_dec)


def kernel(x, W_enc, W_dec, b_enc, b_dec):
    acts = _encode(x, W_enc, b_enc, b_dec)
    return _topk_decode_sc(acts, W_dec, b_dec)
